# baseline (speedup 1.0000x reference)
import functools

import jax
import jax.numpy as jnp
from jax import lax
from jax.experimental import pallas as pl
from jax.experimental.pallas import tpu as pltpu
from jax.experimental.pallas import tpu_sc as plsc

N = 10000
E = 320000
D = 128

NC = 2
NS = 16
NW = NC * NS
EPW = E // NW
KD = 40
NCHD = EPW // KD
K = 80
NCH = EPW // K
NB = 3
NPT = 1000

_mesh = plsc.VectorSubcoreMesh(core_axis_name="c", subcore_axis_name="s")


def _copy_idx_chunk(dst_buf, dst_all, base):
    dst_buf[pl.ds(0, 16)] = dst_all[pl.ds(base, 16)]
    dst_buf[pl.ds(16, 16)] = dst_all[pl.ds(base + 16, 16)]
    dst_buf[pl.ds(24, 16)] = dst_all[pl.ds(base + 24, 16)]


@functools.partial(
    pl.kernel,
    out_type=jax.ShapeDtypeStruct((NC * N,), jnp.float32),
    mesh=_mesh,
    scratch_types=[
        pltpu.VMEM((EPW,), jnp.float32),
        pltpu.VMEM((EPW,), jnp.int32),
        pltpu.VMEM((KD,), jnp.int32),
        pltpu.VMEM((1024,), jnp.float32),
        pltpu.VMEM_SHARED((N,), jnp.float32),
    ],
)
def _sc_deg(ew_hbm, dst_hbm, out_hbm, ew_all, dst_all, dst_buf, zbuf, acc):
    cid = lax.axis_index("c")
    tid = lax.axis_index("s")
    wid = tid * NC + cid
    zv = jnp.zeros((16,), jnp.float32)

    def zb(i, _):
        zbuf[pl.ds(i * 16, 16)] = zv
        return 0

    lax.fori_loop(0, 64, zb, 0)
    @pl.when(tid < 10)
    def _():
        pltpu.sync_copy(zbuf.at[pl.ds(0, 1000)], acc.at[pl.ds(tid * 1000, 1000)])

    pltpu.sync_copy(ew_hbm.at[pl.ds(wid * EPW, EPW)], ew_all)
    pltpu.sync_copy(dst_hbm.at[pl.ds(wid * EPW, EPW)], dst_all)
    plsc.subcore_barrier()

    def chunk(ci, _):
        _copy_idx_chunk(dst_buf, dst_all, ci * KD)
        pltpu.sync_copy(ew_all.at[pl.ds(ci * KD, KD)], acc.at[dst_buf], add=True)
        return 0

    lax.fori_loop(0, NCHD, chunk, 0)
    plsc.subcore_barrier()

    @pl.when(tid < 10)
    def _():
        pltpu.sync_copy(acc.at[pl.ds(tid * 1000, 1000)], zbuf.at[pl.ds(0, 1000)])
        pltpu.sync_copy(zbuf.at[pl.ds(0, 1000)],
                        out_hbm.at[pl.ds(cid * N + tid * 1000, 1000)])


@functools.partial(
    pl.kernel,
    out_type=jax.ShapeDtypeStruct((NC, N, D), jnp.float32),
    mesh=_mesh,
    scratch_types=[
        pltpu.VMEM((EPW,), jnp.int32),
        pltpu.VMEM((1, K), jnp.int32),
        pltpu.VMEM((1, K), jnp.int32),
        pltpu.VMEM((1, K), jnp.int32),
        pltpu.VMEM((1, K), jnp.float32),
        pltpu.VMEM((1, K), jnp.float32),
        pltpu.VMEM((1, K), jnp.float32),
        pltpu.VMEM((K, D), jnp.float32),
        pltpu.VMEM((K, D), jnp.float32),
        pltpu.VMEM((K, D), jnp.float32),
        pltpu.VMEM_SHARED((N, D), jnp.float32),
        pltpu.SemaphoreType.DMA,
        pltpu.SemaphoreType.DMA,
        pltpu.SemaphoreType.DMA,
        pltpu.SemaphoreType.DMA,
        pltpu.SemaphoreType.DMA,
        pltpu.SemaphoreType.DMA,
        pltpu.SemaphoreType.DMA,
        pltpu.SemaphoreType.DMA,
        pltpu.SemaphoreType.DMA,
    ],
)
def _sc_agg(g_hbm, src_hbm, dst_hbm, ew_hbm, out_hbm,
            src_all, ibd0, ibd1, ibd2, ibw0, ibw1, ibw2,
            rows0, rows1, rows2, acc,
            gsem0, gsem1, gsem2, ssem0, ssem1, ssem2, isem0, isem1, isem2):
    cid = lax.axis_index("c")
    tid = lax.axis_index("s")
    wid = tid * NC + cid
    zv = jnp.zeros((16,), jnp.float32)
    rows = (rows0, rows1, rows2)
    ibd = (ibd0, ibd1, ibd2)
    ibw = (ibw0, ibw1, ibw2)
    gsem = (gsem0, gsem1, gsem2)
    ssem = (ssem0, ssem1, ssem2)
    isem = (isem0, isem1, isem2)

    def zb(i, _):
        for c2 in range(D // 16):
            rows0[i, pl.ds(c2 * 16, 16)] = zv
        return 0

    lax.fori_loop(0, K, zb, 0)

    @pl.when(tid < 10)
    def _():
        for j in range(NPT // K):
            pltpu.sync_copy(rows0.at[pl.ds(0, K)],
                            acc.at[pl.ds(tid * NPT + j * K, K)])
        pltpu.sync_copy(rows0.at[pl.ds(0, NPT - (NPT // K) * K)],
                        acc.at[pl.ds(tid * NPT + (NPT // K) * K,
                                     NPT - (NPT // K) * K)])

    pltpu.sync_copy(src_hbm.at[pl.ds(wid * EPW, EPW)], src_all)
    plsc.subcore_barrier()

    def fire_idx(ci, b):
        pltpu.async_copy(dst_hbm.at[wid, ci], ibd[b], isem[b])
        pltpu.async_copy(ew_hbm.at[wid, ci], ibw[b], isem[b])

    def wait_idx(b):
        pltpu.make_async_copy(dst_hbm.at[0, 0], ibd[b], isem[b]).wait()
        pltpu.make_async_copy(ew_hbm.at[0, 0], ibw[b], isem[b]).wait()

    def fire_gather(ci, b):
        pltpu.async_copy(g_hbm.at[src_all.at[pl.ds(ci * K, K)]], rows[b],
                         gsem[b])

    def wait_gather(b):
        pltpu.make_async_copy(g_hbm.at[src_all.at[pl.ds(0, K)]], rows[b],
                              gsem[b]).wait()

    def fire_scatter(b):
        pltpu.async_copy(rows[b], acc.at[ibd[b].at[0]], ssem[b], add=True)

    def wait_scatter(b):
        pltpu.make_async_copy(rows[b], acc.at[ibd[b].at[0]], ssem[b]).wait()

    def scale(b):
        rb = rows[b]
        wb = ibw[b]

        def group(gi, _):
            w16 = wb[0, pl.ds(gi * 16, 16)]
            for j in range(16):
                e = gi * 16 + j
                wv = jnp.full((16,), w16[j], jnp.float32)
                for c2 in range(D // 16):
                    rb[e, pl.ds(c2 * 16, 16)] = rb[e, pl.ds(c2 * 16, 16)] * wv
            return 0

        lax.fori_loop(0, K // 16, group, 0)

    def step(ci, b, b1):
        wait_idx(b)
        wait_gather(b)
        scale(b)
        fire_scatter(b)

        @pl.when(ci + 1 < NCH)
        def _():
            @pl.when(ci >= 2)
            def _():
                wait_scatter(b1)

            fire_gather(ci + 1, b1)
            fire_idx(ci + 1, b1)

    fire_idx(0, 0)
    fire_gather(0, 0)

    def triple(g, _):
        for b in range(NB):
            ci = NB * g + b
            step(ci, b, (b + 1) % NB)
        return 0

    lax.fori_loop(0, NCH // NB, triple, 0)
    for t in range(NCH - (NCH // NB) * NB):
        ci = (NCH // NB) * NB + t
        step(ci, ci % NB, (ci + 1) % NB)
    wait_scatter((NCH - 3) % NB)
    wait_scatter((NCH - 2) % NB)
    wait_scatter((NCH - 1) % NB)
    plsc.subcore_barrier()

    @pl.when(tid < 10)
    def _():
        pltpu.sync_copy(acc.at[pl.ds(tid * NPT, NPT)],
                        out_hbm.at[cid, pl.ds(tid * NPT, NPT)])


_R = 2000


def _dis_block(degp_ref):
    d = degp_ref[0] + degp_ref[1]
    return jnp.where(d > 0, lax.rsqrt(jnp.where(d > 0, d, 1.0)), 0.0)


def _tc1_body(degp_ref, x_ref, w_ref, g_ref):
    dis = _dis_block(degp_ref)
    g_ref[...] = jnp.dot(x_ref[...], w_ref[...],
                         preferred_element_type=jnp.float32) * dis


def _tc2_body(degp_ref, p_ref, b_ref, w_ref, g_ref):
    dis = _dis_block(degp_ref)
    h = (p_ref[0] + p_ref[1]) * dis + b_ref[...]
    g_ref[...] = jnp.dot(h, w_ref[...],
                         preferred_element_type=jnp.float32) * dis


def _tc3_body(degp_ref, q_ref, b_ref, y_ref):
    dis = _dis_block(degp_ref)
    y_ref[...] = jnp.maximum((q_ref[0] + q_ref[1]) * dis + b_ref[...], 0.0)


_degp_spec = pl.BlockSpec((2, _R, 1), lambda i: (0, i, 0))
_mat_spec = pl.BlockSpec((_R, D), lambda i: (i, 0))
_pq_spec = pl.BlockSpec((2, _R, D), lambda i: (0, i, 0))
_w_spec = pl.BlockSpec((D, D), lambda i: (0, 0))
_b_spec = pl.BlockSpec((1, D), lambda i: (0, 0))
_out_nd = jax.ShapeDtypeStruct((N, D), jnp.float32)

_tc1 = pl.pallas_call(
    _tc1_body, grid=(N // _R,),
    in_specs=[_degp_spec, _mat_spec, _w_spec],
    out_specs=_mat_spec, out_shape=_out_nd)

_tc2 = pl.pallas_call(
    _tc2_body, grid=(N // _R,),
    in_specs=[_degp_spec, _pq_spec, _b_spec, _w_spec],
    out_specs=_mat_spec, out_shape=_out_nd)

_tc3 = pl.pallas_call(
    _tc3_body, grid=(N // _R,),
    in_specs=[_degp_spec, _pq_spec, _b_spec],
    out_specs=_mat_spec, out_shape=_out_nd)


def kernel(x, edge_index, edge_weights, W1, b1, W2, b2):
    src = edge_index[0]
    dst = edge_index[1]
    dst4 = dst.reshape(NW, NCH, 1, K)
    ew4 = edge_weights.reshape(NW, NCH, 1, K)
    degp = _sc_deg(edge_weights, dst)
    degp3 = degp.reshape(2, N, 1)
    g1 = _tc1(degp3, x, W1)
    p = _sc_agg(g1, src, dst4, ew4)
    g2 = _tc2(degp3, p, b1.reshape(1, D), W2)
    q = _sc_agg(g2, src, dst4, ew4)
    return _tc3(degp3, q, b2.reshape(1, D))


# speedup vs baseline: 1.0001x; 1.0001x over previous
import functools

import jax
import jax.numpy as jnp
from jax import lax
from jax.experimental import pallas as pl
from jax.experimental.pallas import tpu as pltpu
from jax.experimental.pallas import tpu_sc as plsc

N = 10000
E = 320000
D = 128

NC = 2
NS = 16
NW = NC * NS
EPW = E // NW
KD = 40
NCHD = EPW // KD
K = 80
NCH = EPW // K
NB = 3
NPT = 1000

_mesh = plsc.VectorSubcoreMesh(core_axis_name="c", subcore_axis_name="s")


def _copy_idx_chunk(dst_buf, dst_all, base):
    dst_buf[pl.ds(0, 16)] = dst_all[pl.ds(base, 16)]
    dst_buf[pl.ds(16, 16)] = dst_all[pl.ds(base + 16, 16)]
    dst_buf[pl.ds(24, 16)] = dst_all[pl.ds(base + 24, 16)]


@functools.partial(
    pl.kernel,
    out_type=jax.ShapeDtypeStruct((NC * N,), jnp.float32),
    mesh=_mesh,
    scratch_types=[
        pltpu.VMEM((EPW,), jnp.float32),
        pltpu.VMEM((EPW,), jnp.int32),
        pltpu.VMEM((KD,), jnp.int32),
        pltpu.VMEM((1024,), jnp.float32),
        pltpu.VMEM_SHARED((N,), jnp.float32),
    ],
)
def _sc_deg(ew_hbm, dst_hbm, out_hbm, ew_all, dst_all, dst_buf, zbuf, acc):
    cid = lax.axis_index("c")
    tid = lax.axis_index("s")
    wid = tid * NC + cid
    zv = jnp.zeros((16,), jnp.float32)

    def zb(i, _):
        zbuf[pl.ds(i * 16, 16)] = zv
        return 0

    lax.fori_loop(0, 64, zb, 0)
    @pl.when(tid < 10)
    def _():
        pltpu.sync_copy(zbuf.at[pl.ds(0, 1000)], acc.at[pl.ds(tid * 1000, 1000)])

    pltpu.sync_copy(ew_hbm.at[pl.ds(wid * EPW, EPW)], ew_all)
    pltpu.sync_copy(dst_hbm.at[pl.ds(wid * EPW, EPW)], dst_all)
    plsc.subcore_barrier()

    def chunk(ci, _):
        _copy_idx_chunk(dst_buf, dst_all, ci * KD)
        pltpu.sync_copy(ew_all.at[pl.ds(ci * KD, KD)], acc.at[dst_buf], add=True)
        return 0

    lax.fori_loop(0, NCHD, chunk, 0)
    plsc.subcore_barrier()

    @pl.when(tid < 10)
    def _():
        pltpu.sync_copy(acc.at[pl.ds(tid * 1000, 1000)], zbuf.at[pl.ds(0, 1000)])
        pltpu.sync_copy(zbuf.at[pl.ds(0, 1000)],
                        out_hbm.at[pl.ds(cid * N + tid * 1000, 1000)])


@functools.partial(
    pl.kernel,
    out_type=jax.ShapeDtypeStruct((NC, N, D), jnp.float32),
    mesh=_mesh,
    scratch_types=[
        pltpu.VMEM((EPW,), jnp.int32),
        pltpu.VMEM((1, K), jnp.int32),
        pltpu.VMEM((1, K), jnp.int32),
        pltpu.VMEM((1, K), jnp.int32),
        pltpu.VMEM((1, K), jnp.float32),
        pltpu.VMEM((1, K), jnp.float32),
        pltpu.VMEM((1, K), jnp.float32),
        pltpu.VMEM((K, D), jnp.float32),
        pltpu.VMEM((K, D), jnp.float32),
        pltpu.VMEM((K, D), jnp.float32),
        pltpu.VMEM((K,), jnp.int32),
        pltpu.VMEM((K,), jnp.int32),
        pltpu.VMEM((K,), jnp.int32),
        pltpu.VMEM_SHARED((N, D), jnp.float32),
        pltpu.SemaphoreType.DMA,
        pltpu.SemaphoreType.DMA,
        pltpu.SemaphoreType.DMA,
        pltpu.SemaphoreType.DMA,
        pltpu.SemaphoreType.DMA,
        pltpu.SemaphoreType.DMA,
        pltpu.SemaphoreType.DMA,
        pltpu.SemaphoreType.DMA,
        pltpu.SemaphoreType.DMA,
    ],
)
def _sc_agg(g_hbm, src_hbm, dst_hbm, ew_hbm, out_hbm,
            src_all, ibd0, ibd1, ibd2, ibw0, ibw1, ibw2,
            rows0, rows1, rows2, dstb0, dstb1, dstb2, acc,
            gsem0, gsem1, gsem2, ssem0, ssem1, ssem2, isem0, isem1, isem2):
    cid = lax.axis_index("c")
    tid = lax.axis_index("s")
    wid = tid * NC + cid
    zv = jnp.zeros((16,), jnp.float32)
    rows = (rows0, rows1, rows2)
    ibd = (ibd0, ibd1, ibd2)
    ibw = (ibw0, ibw1, ibw2)
    dstb = (dstb0, dstb1, dstb2)
    gsem = (gsem0, gsem1, gsem2)
    ssem = (ssem0, ssem1, ssem2)
    isem = (isem0, isem1, isem2)

    def zb(i, _):
        for c2 in range(D // 16):
            rows0[i, pl.ds(c2 * 16, 16)] = zv
        return 0

    lax.fori_loop(0, K, zb, 0)

    @pl.when(tid < 10)
    def _():
        for j in range(NPT // K):
            pltpu.sync_copy(rows0.at[pl.ds(0, K)],
                            acc.at[pl.ds(tid * NPT + j * K, K)])
        pltpu.sync_copy(rows0.at[pl.ds(0, NPT - (NPT // K) * K)],
                        acc.at[pl.ds(tid * NPT + (NPT // K) * K,
                                     NPT - (NPT // K) * K)])

    pltpu.sync_copy(src_hbm.at[pl.ds(wid * EPW, EPW)], src_all)
    plsc.subcore_barrier()

    def fire_idx(ci, b):
        pltpu.async_copy(dst_hbm.at[wid, ci], ibd[b], isem[b])
        pltpu.async_copy(ew_hbm.at[wid, ci], ibw[b], isem[b])

    def wait_idx(b):
        pltpu.make_async_copy(dst_hbm.at[0, 0], ibd[b], isem[b]).wait()
        pltpu.make_async_copy(ew_hbm.at[0, 0], ibw[b], isem[b]).wait()

    def fire_gather(ci, b):
        pltpu.async_copy(g_hbm.at[src_all.at[pl.ds(ci * K, K)]], rows[b],
                         gsem[b])

    def wait_gather(b):
        pltpu.make_async_copy(g_hbm.at[src_all.at[pl.ds(0, K)]], rows[b],
                              gsem[b]).wait()

    def fire_scatter(b):
        pltpu.async_copy(rows[b], acc.at[dstb[b]], ssem[b], add=True)

    def wait_scatter(b):
        pltpu.make_async_copy(rows[b], acc.at[dstb[b]], ssem[b]).wait()

    def scale(b):
        rb = rows[b]
        wb = ibw[b]

        def group(gi, _):
            w16 = wb[0, pl.ds(gi * 16, 16)]
            for j in range(16):
                e = gi * 16 + j
                wv = jnp.full((16,), w16[j], jnp.float32)
                for c2 in range(D // 16):
                    rb[e, pl.ds(c2 * 16, 16)] = rb[e, pl.ds(c2 * 16, 16)] * wv
            return 0

        lax.fori_loop(0, K // 16, group, 0)

    def step(ci, b, b1, b2):
        wait_idx(b)
        for i in range(K // 16):
            dstb[b][pl.ds(i * 16, 16)] = ibd[b][0, pl.ds(i * 16, 16)]
        wait_gather(b)
        scale(b)
        fire_scatter(b)

        @pl.when(ci + 1 < NCH)
        def _():
            @pl.when(ci >= 2)
            def _():
                wait_scatter(b1)

            fire_gather(ci + 1, b1)

        @pl.when(ci + 2 < NCH)
        def _():
            fire_idx(ci + 2, b2)

    fire_idx(0, 0)
    fire_idx(1, 1)
    fire_gather(0, 0)

    def triple(g, _):
        for b in range(NB):
            ci = NB * g + b
            step(ci, b, (b + 1) % NB, (b + 2) % NB)
        return 0

    lax.fori_loop(0, NCH // NB, triple, 0)
    for t in range(NCH - (NCH // NB) * NB):
        ci = (NCH // NB) * NB + t
        step(ci, ci % NB, (ci + 1) % NB, (ci + 2) % NB)
    wait_scatter((NCH - 3) % NB)
    wait_scatter((NCH - 2) % NB)
    wait_scatter((NCH - 1) % NB)
    plsc.subcore_barrier()

    @pl.when(tid < 10)
    def _():
        pltpu.sync_copy(acc.at[pl.ds(tid * NPT, NPT)],
                        out_hbm.at[cid, pl.ds(tid * NPT, NPT)])


_R = 2000


def _dis_block(degp_ref):
    d = degp_ref[0] + degp_ref[1]
    return jnp.where(d > 0, lax.rsqrt(jnp.where(d > 0, d, 1.0)), 0.0)


def _tc1_body(degp_ref, x_ref, w_ref, g_ref):
    dis = _dis_block(degp_ref)
    g_ref[...] = jnp.dot(x_ref[...], w_ref[...],
                         preferred_element_type=jnp.float32) * dis


def _tc2_body(degp_ref, p_ref, b_ref, w_ref, g_ref):
    dis = _dis_block(degp_ref)
    h = (p_ref[0] + p_ref[1]) * dis + b_ref[...]
    g_ref[...] = jnp.dot(h, w_ref[...],
                         preferred_element_type=jnp.float32) * dis


def _tc3_body(degp_ref, q_ref, b_ref, y_ref):
    dis = _dis_block(degp_ref)
    y_ref[...] = jnp.maximum((q_ref[0] + q_ref[1]) * dis + b_ref[...], 0.0)


_degp_spec = pl.BlockSpec((2, _R, 1), lambda i: (0, i, 0))
_mat_spec = pl.BlockSpec((_R, D), lambda i: (i, 0))
_pq_spec = pl.BlockSpec((2, _R, D), lambda i: (0, i, 0))
_w_spec = pl.BlockSpec((D, D), lambda i: (0, 0))
_b_spec = pl.BlockSpec((1, D), lambda i: (0, 0))
_out_nd = jax.ShapeDtypeStruct((N, D), jnp.float32)

_tc1 = pl.pallas_call(
    _tc1_body, grid=(N // _R,),
    in_specs=[_degp_spec, _mat_spec, _w_spec],
    out_specs=_mat_spec, out_shape=_out_nd)

_tc2 = pl.pallas_call(
    _tc2_body, grid=(N // _R,),
    in_specs=[_degp_spec, _pq_spec, _b_spec, _w_spec],
    out_specs=_mat_spec, out_shape=_out_nd)

_tc3 = pl.pallas_call(
    _tc3_body, grid=(N // _R,),
    in_specs=[_degp_spec, _pq_spec, _b_spec],
    out_specs=_mat_spec, out_shape=_out_nd)


def kernel(x, edge_index, edge_weights, W1, b1, W2, b2):
    src = edge_index[0]
    dst = edge_index[1]
    dst4 = dst.reshape(NW, NCH, 1, K)
    ew4 = edge_weights.reshape(NW, NCH, 1, K)
    degp = _sc_deg(edge_weights, dst)
    degp3 = degp.reshape(2, N, 1)
    g1 = _tc1(degp3, x, W1)
    p = _sc_agg(g1, src, dst4, ew4)
    g2 = _tc2(degp3, p, b1.reshape(1, D), W2)
    q = _sc_agg(g2, src, dst4, ew4)
    return _tc3(degp3, q, b2.reshape(1, D))


# speedup vs baseline: 1.0765x; 1.0764x over previous
import functools

import jax
import jax.numpy as jnp
from jax import lax
from jax.experimental import pallas as pl
from jax.experimental.pallas import tpu as pltpu
from jax.experimental.pallas import tpu_sc as plsc

N = 10000
E = 320000
D = 128

NC = 2
NS = 16
NW = NC * NS
EPW = E // NW
KD = 40
NCHD = EPW // KD
K = 40
NCH = EPW // K
NPT = 1000

_mesh = plsc.VectorSubcoreMesh(core_axis_name="c", subcore_axis_name="s")


def _copy_idx_chunk(dst_buf, dst_all, base):
    dst_buf[pl.ds(0, 16)] = dst_all[pl.ds(base, 16)]
    dst_buf[pl.ds(16, 16)] = dst_all[pl.ds(base + 16, 16)]
    dst_buf[pl.ds(24, 16)] = dst_all[pl.ds(base + 24, 16)]


@functools.partial(
    pl.kernel,
    out_type=jax.ShapeDtypeStruct((NC * N,), jnp.float32),
    mesh=_mesh,
    scratch_types=[
        pltpu.VMEM((EPW,), jnp.float32),
        pltpu.VMEM((EPW,), jnp.int32),
        pltpu.VMEM((KD,), jnp.int32),
        pltpu.VMEM((1024,), jnp.float32),
        pltpu.VMEM_SHARED((N,), jnp.float32),
    ],
)
def _sc_deg(ew_hbm, dst_hbm, out_hbm, ew_all, dst_all, dst_buf, zbuf, acc):
    cid = lax.axis_index("c")
    tid = lax.axis_index("s")
    wid = tid * NC + cid
    zv = jnp.zeros((16,), jnp.float32)

    def zb(i, _):
        zbuf[pl.ds(i * 16, 16)] = zv
        return 0

    lax.fori_loop(0, 64, zb, 0)
    @pl.when(tid < 10)
    def _():
        pltpu.sync_copy(zbuf.at[pl.ds(0, 1000)], acc.at[pl.ds(tid * 1000, 1000)])

    pltpu.sync_copy(ew_hbm.at[pl.ds(wid * EPW, EPW)], ew_all)
    pltpu.sync_copy(dst_hbm.at[pl.ds(wid * EPW, EPW)], dst_all)
    plsc.subcore_barrier()

    def chunk(ci, _):
        _copy_idx_chunk(dst_buf, dst_all, ci * KD)
        pltpu.sync_copy(ew_all.at[pl.ds(ci * KD, KD)], acc.at[dst_buf], add=True)
        return 0

    lax.fori_loop(0, NCHD, chunk, 0)
    plsc.subcore_barrier()

    @pl.when(tid < 10)
    def _():
        pltpu.sync_copy(acc.at[pl.ds(tid * 1000, 1000)], zbuf.at[pl.ds(0, 1000)])
        pltpu.sync_copy(zbuf.at[pl.ds(0, 1000)],
                        out_hbm.at[pl.ds(cid * N + tid * 1000, 1000)])


@functools.partial(
    pl.kernel,
    out_type=jax.ShapeDtypeStruct((NC, N, D), jnp.float32),
    mesh=_mesh,
    scratch_types=[
        pltpu.VMEM((EPW,), jnp.int32),
        pltpu.VMEM((EPW + 16,), jnp.float32),
        pltpu.VMEM((EPW,), jnp.int32),
        pltpu.VMEM((K,), jnp.int32),
        pltpu.VMEM((K,), jnp.int32),
        pltpu.VMEM((K, D), jnp.float32),
        pltpu.VMEM((K, D), jnp.float32),
        pltpu.VMEM_SHARED((N, D), jnp.float32),
        pltpu.SemaphoreType.DMA,
        pltpu.SemaphoreType.DMA,
        pltpu.SemaphoreType.DMA,
        pltpu.SemaphoreType.DMA,
    ],
)
def _sc_agg(g_hbm, src_hbm, ew_hbm, dst_hbm, out_hbm,
            src_all, ew_all, dst_all, dstb0, dstb1, rows0, rows1, acc,
            gsem0, gsem1, ssem0, ssem1):
    cid = lax.axis_index("c")
    tid = lax.axis_index("s")
    wid = tid * NC + cid
    zv = jnp.zeros((16,), jnp.float32)
    rows = (rows0, rows1)
    dstb = (dstb0, dstb1)
    gsem = (gsem0, gsem1)
    ssem = (ssem0, ssem1)

    def zb(i, _):
        for c2 in range(D // 16):
            rows0[i, pl.ds(c2 * 16, 16)] = zv
        return 0

    lax.fori_loop(0, K, zb, 0)

    @pl.when(tid < 10)
    def _():
        for j in range(NPT // K):
            pltpu.sync_copy(rows0.at[pl.ds(0, K)],
                            acc.at[pl.ds(tid * NPT + j * K, K)])

    pltpu.sync_copy(src_hbm.at[pl.ds(wid * EPW, EPW)], src_all)
    pltpu.sync_copy(ew_hbm.at[pl.ds(wid * EPW, EPW)], ew_all.at[pl.ds(0, EPW)])
    pltpu.sync_copy(dst_hbm.at[pl.ds(wid * EPW, EPW)], dst_all)
    plsc.subcore_barrier()

    def fire_gather(ci, b):
        pltpu.async_copy(g_hbm.at[src_all.at[pl.ds(ci * K, K)]], rows[b],
                         gsem[b])

    def wait_gather(b):
        pltpu.make_async_copy(g_hbm.at[src_all.at[pl.ds(0, K)]], rows[b],
                              gsem[b]).wait()

    def fire_scatter(b):
        pltpu.async_copy(rows[b], acc.at[dstb[b]], ssem[b], add=True)

    def wait_scatter(b):
        pltpu.make_async_copy(rows[b], acc.at[dstb[b]], ssem[b]).wait()

    def scale(ci, b):
        rb = rows[b]

        def group(gi, _):
            w16 = ew_all[pl.ds(ci * K + gi * 16, 16)]
            for j in range(16):
                e = gi * 16 + j
                wv = jnp.full((16,), w16[j], jnp.float32)
                for c2 in range(D // 16):
                    rb[e, pl.ds(c2 * 16, 16)] = rb[e, pl.ds(c2 * 16, 16)] * wv
            return 0

        lax.fori_loop(0, 2, group, 0)
        w16 = ew_all[pl.ds(ci * K + 32, 16)]
        for j in range(8):
            e = 32 + j
            wv = jnp.full((16,), w16[j], jnp.float32)
            for c2 in range(D // 16):
                rb[e, pl.ds(c2 * 16, 16)] = rb[e, pl.ds(c2 * 16, 16)] * wv

    fire_gather(0, 0)

    def pair(g, _):
        for b in range(2):
            ci = 2 * g + b

            @pl.when(ci >= 1)
            def _():
                wait_scatter(1 - b)

            @pl.when(ci + 1 < NCH)
            def _():
                fire_gather(ci + 1, 1 - b)

            wait_gather(b)
            _copy_idx_chunk(dstb[b], dst_all, ci * K)
            scale(ci, b)
            fire_scatter(b)
        return 0

    lax.fori_loop(0, NCH // 2, pair, 0)
    wait_scatter(1)
    plsc.subcore_barrier()

    @pl.when(tid < 10)
    def _():
        pltpu.sync_copy(acc.at[pl.ds(tid * NPT, NPT)],
                        out_hbm.at[cid, pl.ds(tid * NPT, NPT)])


_R = 2000


def _dis_block(degp_ref):
    d = degp_ref[0] + degp_ref[1]
    return jnp.where(d > 0, lax.rsqrt(jnp.where(d > 0, d, 1.0)), 0.0)


def _tc1_body(degp_ref, x_ref, w_ref, g_ref):
    dis = _dis_block(degp_ref)
    g_ref[...] = jnp.dot(x_ref[...], w_ref[...],
                         preferred_element_type=jnp.float32) * dis


def _tc2_body(degp_ref, p_ref, b_ref, w_ref, g_ref):
    dis = _dis_block(degp_ref)
    h = (p_ref[0] + p_ref[1]) * dis + b_ref[...]
    g_ref[...] = jnp.dot(h, w_ref[...],
                         preferred_element_type=jnp.float32) * dis


def _tc3_body(degp_ref, q_ref, b_ref, y_ref):
    dis = _dis_block(degp_ref)
    y_ref[...] = jnp.maximum((q_ref[0] + q_ref[1]) * dis + b_ref[...], 0.0)


_degp_spec = pl.BlockSpec((2, _R, 1), lambda i: (0, i, 0))
_mat_spec = pl.BlockSpec((_R, D), lambda i: (i, 0))
_pq_spec = pl.BlockSpec((2, _R, D), lambda i: (0, i, 0))
_w_spec = pl.BlockSpec((D, D), lambda i: (0, 0))
_b_spec = pl.BlockSpec((1, D), lambda i: (0, 0))
_out_nd = jax.ShapeDtypeStruct((N, D), jnp.float32)

_tc1 = pl.pallas_call(
    _tc1_body, grid=(N // _R,),
    in_specs=[_degp_spec, _mat_spec, _w_spec],
    out_specs=_mat_spec, out_shape=_out_nd)

_tc2 = pl.pallas_call(
    _tc2_body, grid=(N // _R,),
    in_specs=[_degp_spec, _pq_spec, _b_spec, _w_spec],
    out_specs=_mat_spec, out_shape=_out_nd)

_tc3 = pl.pallas_call(
    _tc3_body, grid=(N // _R,),
    in_specs=[_degp_spec, _pq_spec, _b_spec],
    out_specs=_mat_spec, out_shape=_out_nd)


def kernel(x, edge_index, edge_weights, W1, b1, W2, b2):
    src = edge_index[0]
    dst = edge_index[1]
    degp = _sc_deg(edge_weights, dst)
    degp3 = degp.reshape(2, N, 1)
    g1 = _tc1(degp3, x, W1)
    p = _sc_agg(g1, src, edge_weights, dst)
    g2 = _tc2(degp3, p, b1.reshape(1, D), W2)
    q = _sc_agg(g2, src, edge_weights, dst)
    return _tc3(degp3, q, b2.reshape(1, D))


# speedup vs baseline: 1.1021x; 1.0238x over previous
import functools

import jax
import jax.numpy as jnp
from jax import lax
from jax.experimental import pallas as pl
from jax.experimental.pallas import tpu as pltpu
from jax.experimental.pallas import tpu_sc as plsc

N = 10000
E = 320000
D = 128

NC = 2
NS = 16
NW = NC * NS
EPW = E // NW
KD = 80
NCHD = EPW // KD
K = 40
NCH = EPW // K
NPT = 1000

_mesh = plsc.VectorSubcoreMesh(core_axis_name="c", subcore_axis_name="s")


def _copy_idx_chunk(dst_buf, dst_all, base, k):
    n_full = k // 16
    for i in range(n_full):
        dst_buf[pl.ds(i * 16, 16)] = dst_all[pl.ds(base + i * 16, 16)]
    if k % 16:
        off = k - 16
        dst_buf[pl.ds(off, 16)] = dst_all[pl.ds(base + off, 16)]


@functools.partial(
    pl.kernel,
    out_type=jax.ShapeDtypeStruct((NC * N,), jnp.float32),
    mesh=_mesh,
    scratch_types=[
        pltpu.VMEM((EPW,), jnp.float32),
        pltpu.VMEM((EPW,), jnp.int32),
        pltpu.VMEM((KD,), jnp.int32),
        pltpu.VMEM((1024,), jnp.float32),
        pltpu.VMEM_SHARED((N,), jnp.float32),
    ],
)
def _sc_deg(ew_hbm, dst_hbm, out_hbm, ew_all, dst_all, dst_buf, zbuf, acc):
    cid = lax.axis_index("c")
    tid = lax.axis_index("s")
    wid = tid * NC + cid
    zv = jnp.zeros((16,), jnp.float32)

    def zb(i, _):
        zbuf[pl.ds(i * 16, 16)] = zv
        return 0

    lax.fori_loop(0, 64, zb, 0)
    @pl.when(tid < 10)
    def _():
        pltpu.sync_copy(zbuf.at[pl.ds(0, 1000)], acc.at[pl.ds(tid * 1000, 1000)])

    pltpu.sync_copy(ew_hbm.at[pl.ds(wid * EPW, EPW)], ew_all)
    pltpu.sync_copy(dst_hbm.at[pl.ds(wid * EPW, EPW)], dst_all)
    plsc.subcore_barrier()

    def chunk(ci, _):
        _copy_idx_chunk(dst_buf, dst_all, ci * KD, KD)
        pltpu.sync_copy(ew_all.at[pl.ds(ci * KD, KD)], acc.at[dst_buf], add=True)
        return 0

    lax.fori_loop(0, NCHD, chunk, 0)
    plsc.subcore_barrier()

    @pl.when(tid < 10)
    def _():
        pltpu.sync_copy(acc.at[pl.ds(tid * 1000, 1000)], zbuf.at[pl.ds(0, 1000)])
        pltpu.sync_copy(zbuf.at[pl.ds(0, 1000)],
                        out_hbm.at[pl.ds(cid * N + tid * 1000, 1000)])


@functools.partial(
    pl.kernel,
    out_type=jax.ShapeDtypeStruct((NC, N, D), jnp.float32),
    mesh=_mesh,
    scratch_types=[
        pltpu.VMEM((EPW,), jnp.int32),
        pltpu.VMEM((EPW + 16,), jnp.float32),
        pltpu.VMEM((EPW,), jnp.int32),
        pltpu.VMEM((K,), jnp.int32),
        pltpu.VMEM((K,), jnp.int32),
        pltpu.VMEM((K, D), jnp.float32),
        pltpu.VMEM((K, D), jnp.float32),
        pltpu.VMEM_SHARED((N, D), jnp.float32),
        pltpu.SemaphoreType.DMA,
        pltpu.SemaphoreType.DMA,
        pltpu.SemaphoreType.DMA,
        pltpu.SemaphoreType.DMA,
    ],
)
def _sc_agg(g_hbm, src_hbm, ew_hbm, dst_hbm, out_hbm,
            src_all, ew_all, dst_all, dstb0, dstb1, rows0, rows1, acc,
            gsem0, gsem1, ssem0, ssem1):
    cid = lax.axis_index("c")
    tid = lax.axis_index("s")
    wid = tid * NC + cid
    zv = jnp.zeros((16,), jnp.float32)
    rows = (rows0, rows1)
    dstb = (dstb0, dstb1)
    gsem = (gsem0, gsem1)
    ssem = (ssem0, ssem1)

    def zb(i, _):
        for c2 in range(D // 16):
            rows0[i, pl.ds(c2 * 16, 16)] = zv
        return 0

    lax.fori_loop(0, K, zb, 0)

    @pl.when(tid < 10)
    def _():
        for j in range(NPT // K):
            pltpu.sync_copy(rows0.at[pl.ds(0, K)],
                            acc.at[pl.ds(tid * NPT + j * K, K)])

    pltpu.sync_copy(src_hbm.at[pl.ds(wid * EPW, EPW)], src_all)
    pltpu.sync_copy(ew_hbm.at[pl.ds(wid * EPW, EPW)], ew_all.at[pl.ds(0, EPW)])
    pltpu.sync_copy(dst_hbm.at[pl.ds(wid * EPW, EPW)], dst_all)
    plsc.subcore_barrier()

    def fire_gather(ci, b):
        pltpu.async_copy(g_hbm.at[src_all.at[pl.ds(ci * K, K)]], rows[b],
                         gsem[b])

    def wait_gather(b):
        pltpu.make_async_copy(g_hbm.at[src_all.at[pl.ds(0, K)]], rows[b],
                              gsem[b]).wait()

    def fire_scatter(b):
        pltpu.async_copy(rows[b], acc.at[dstb[b]], ssem[b], add=True)

    def wait_scatter(b):
        pltpu.make_async_copy(rows[b], acc.at[dstb[b]], ssem[b]).wait()

    def scale(ci, b):
        rb = rows[b]

        def group(gi, _):
            w16 = ew_all[pl.ds(ci * K + gi * 16, 16)]
            for j in range(16):
                e = gi * 16 + j
                wv = jnp.full((16,), w16[j], jnp.float32)
                for c2 in range(D // 16):
                    rb[e, pl.ds(c2 * 16, 16)] = rb[e, pl.ds(c2 * 16, 16)] * wv
            return 0

        lax.fori_loop(0, 2, group, 0)
        w16 = ew_all[pl.ds(ci * K + 32, 16)]
        for j in range(8):
            e = 32 + j
            wv = jnp.full((16,), w16[j], jnp.float32)
            for c2 in range(D // 16):
                rb[e, pl.ds(c2 * 16, 16)] = rb[e, pl.ds(c2 * 16, 16)] * wv

    fire_gather(0, 0)

    def pair(g, _):
        for b in range(2):
            ci = 2 * g + b

            @pl.when(ci >= 1)
            def _():
                wait_scatter(1 - b)

            @pl.when(ci + 1 < NCH)
            def _():
                fire_gather(ci + 1, 1 - b)

            _copy_idx_chunk(dstb[b], dst_all, ci * K, K)
            wait_gather(b)
            scale(ci, b)
            fire_scatter(b)
        return 0

    lax.fori_loop(0, NCH // 2, pair, 0)
    wait_scatter(1)
    plsc.subcore_barrier()

    @pl.when(tid < 10)
    def _():
        pltpu.sync_copy(acc.at[pl.ds(tid * NPT, NPT)],
                        out_hbm.at[cid, pl.ds(tid * NPT, NPT)])


_R = 2000


def _dis_block(degp_ref):
    d = degp_ref[0] + degp_ref[1]
    return jnp.where(d > 0, lax.rsqrt(jnp.where(d > 0, d, 1.0)), 0.0)


def _tc1_body(degp_ref, x_ref, w_ref, g_ref):
    dis = _dis_block(degp_ref)
    g_ref[...] = jnp.dot(x_ref[...], w_ref[...],
                         preferred_element_type=jnp.float32) * dis


def _tc2_body(degp_ref, p_ref, b_ref, w_ref, g_ref):
    dis = _dis_block(degp_ref)
    h = (p_ref[0] + p_ref[1]) * dis + b_ref[...]
    g_ref[...] = jnp.dot(h, w_ref[...],
                         preferred_element_type=jnp.float32) * dis


def _tc3_body(degp_ref, q_ref, b_ref, y_ref):
    dis = _dis_block(degp_ref)
    y_ref[...] = jnp.maximum((q_ref[0] + q_ref[1]) * dis + b_ref[...], 0.0)


_degp_spec = pl.BlockSpec((2, _R, 1), lambda i: (0, i, 0))
_mat_spec = pl.BlockSpec((_R, D), lambda i: (i, 0))
_pq_spec = pl.BlockSpec((2, _R, D), lambda i: (0, i, 0))
_w_spec = pl.BlockSpec((D, D), lambda i: (0, 0))
_b_spec = pl.BlockSpec((1, D), lambda i: (0, 0))
_out_nd = jax.ShapeDtypeStruct((N, D), jnp.float32)

_tc1 = pl.pallas_call(
    _tc1_body, grid=(N // _R,),
    in_specs=[_degp_spec, _mat_spec, _w_spec],
    out_specs=_mat_spec, out_shape=_out_nd)

_tc2 = pl.pallas_call(
    _tc2_body, grid=(N // _R,),
    in_specs=[_degp_spec, _pq_spec, _b_spec, _w_spec],
    out_specs=_mat_spec, out_shape=_out_nd)

_tc3 = pl.pallas_call(
    _tc3_body, grid=(N // _R,),
    in_specs=[_degp_spec, _pq_spec, _b_spec],
    out_specs=_mat_spec, out_shape=_out_nd)


def kernel(x, edge_index, edge_weights, W1, b1, W2, b2):
    src = edge_index[0]
    dst = edge_index[1]
    degp = _sc_deg(edge_weights, dst)
    degp3 = degp.reshape(2, N, 1)
    g1 = _tc1(degp3, x, W1)
    p = _sc_agg(g1, src, edge_weights, dst)
    g2 = _tc2(degp3, p, b1.reshape(1, D), W2)
    q = _sc_agg(g2, src, edge_weights, dst)
    return _tc3(degp3, q, b2.reshape(1, D))
